# baseline (speedup 1.0000x reference)
import functools

import jax
import jax.numpy as jnp
import numpy as np
from jax import lax
from jax.experimental import pallas as pl
from jax.experimental.pallas import tpu as pltpu
from jax.experimental.pallas import tpu_sc as plsc

_info = plsc.get_sparse_core_info()
_NC, _NS = _info.num_cores, _info.num_subcores
_NW = _NC * _NS

_LANES = 128
_IOTA16 = np.arange(16, dtype=np.int32)


def _embed_kernel(n_blocks, idx_hbm, table_hbm, o5_hbm,
                  idx_v, g0, g1, ob0, ob1, gsem0, gsem1, wsem0, wsem1):
    wid = lax.axis_index("s") * _NC + lax.axis_index("c")
    pltpu.sync_copy(idx_hbm.at[wid], idx_v)

    def issue(j, buf, sem):
        pltpu.async_copy(table_hbm.at[idx_v.at[j]], buf, sem)

    def drain_gather(buf, sem):
        pltpu.make_async_copy(table_hbm.at[idx_v.at[0]], buf, sem).wait()

    def transpose_block(gbuf, obuf):
        iota = lax.iota(jnp.int32, 16)
        zero = iota * 0
        lanes = [iota + (16 * lb) for lb in range(8)]
        for d in range(32):
            dv = zero + d
            for lb in range(8):
                vals = plsc.load_gather(gbuf, [lanes[lb], dv])
                obuf[d // 8, d % 8, pl.ds(16 * lb, 16)] = vals

    def write_out(j, obuf, wsem):
        beta = wid * n_blocks + j
        s = beta // 128
        k = beta % 128
        for tr in range(4):
            pltpu.async_copy(obuf.at[tr], o5_hbm.at[s, tr, k], wsem)

    def drain_write(obuf, wsem):
        for tr in range(4):
            pltpu.make_async_copy(obuf.at[tr], o5_hbm.at[0, tr, 0],
                                  wsem).wait()

    issue(0, g0, gsem0)

    def body(i, carry):
        issue(2 * i + 1, g1, gsem1)
        drain_gather(g0, gsem0)

        @pl.when(i > 0)
        def _():
            drain_write(ob0, wsem0)

        transpose_block(g0, ob0)
        write_out(2 * i, ob0, wsem0)

        @pl.when(i < n_blocks // 2 - 1)
        def _():
            issue(2 * i + 2, g0, gsem0)

        drain_gather(g1, gsem1)

        @pl.when(i > 0)
        def _():
            drain_write(ob1, wsem1)

        transpose_block(g1, ob1)
        write_out(2 * i + 1, ob1, wsem1)
        return carry

    lax.fori_loop(0, n_blocks // 2, body, 0)
    drain_write(ob0, wsem0)
    drain_write(ob1, wsem1)


def kernel(x, w):
    b, s_len = x.shape
    d = w.shape[1]
    n_blk_rows = b // _LANES
    total_blocks = s_len * n_blk_rows
    assert total_blocks % (2 * _NW) == 0
    n_blocks = total_blocks // _NW
    idx = x.astype(jnp.int32).T.reshape(_NW, n_blocks, _LANES)

    mesh = plsc.VectorSubcoreMesh(core_axis_name="c", subcore_axis_name="s")
    k = functools.partial(
        pl.kernel,
        mesh=mesh,
        out_type=jax.ShapeDtypeStruct((s_len, d // 8, n_blk_rows, 8, _LANES),
                                      jnp.float32),
        scratch_types=[
            pltpu.VMEM((n_blocks, _LANES), jnp.int32),
            pltpu.VMEM((_LANES, d), jnp.float32),
            pltpu.VMEM((_LANES, d), jnp.float32),
            pltpu.VMEM((d // 8, 8, _LANES), jnp.float32),
            pltpu.VMEM((d // 8, 8, _LANES), jnp.float32),
            pltpu.SemaphoreType.DMA,
            pltpu.SemaphoreType.DMA,
            pltpu.SemaphoreType.DMA,
            pltpu.SemaphoreType.DMA,
        ],
        compiler_params=pltpu.CompilerParams(use_tc_tiling_on_sc=False,
                                             needs_layout_passes=False),
    )(functools.partial(_embed_kernel, n_blocks))
    o5 = k(idx, w)
    return o5.transpose(2, 4, 0, 1, 3).reshape(b, s_len, d)


# speedup vs baseline: 1.0687x; 1.0687x over previous
import functools

import jax
import jax.numpy as jnp
from jax import lax
from jax.experimental import pallas as pl
from jax.experimental.pallas import tpu as pltpu
from jax.experimental.pallas import tpu_sc as plsc

_info = plsc.get_sparse_core_info()
_NC, _NS = _info.num_cores, _info.num_subcores
_NW = _NC * _NS

_LANES = 128
_NBUF = 4


def _embed_kernel(n_blocks, idx_hbm, table_hbm, o5_hbm, idx_v,
                  g0, g1, g2, g3, ob0, ob1, ob2, ob3,
                  gs0, gs1, gs2, gs3, ws0, ws1, ws2, ws3):
    wid = lax.axis_index("s") * _NC + lax.axis_index("c")
    pltpu.sync_copy(idx_hbm.at[wid], idx_v)
    gbufs = (g0, g1, g2, g3)
    obufs = (ob0, ob1, ob2, ob3)
    gsems = (gs0, gs1, gs2, gs3)
    wsems = (ws0, ws1, ws2, ws3)

    def issue(j, q):
        pltpu.async_copy(table_hbm.at[idx_v.at[j]], gbufs[q], gsems[q])

    def drain_gather(q):
        pltpu.make_async_copy(table_hbm.at[idx_v.at[0]], gbufs[q],
                              gsems[q]).wait()

    def transpose_block(gbuf, obuf):
        iota = lax.iota(jnp.int32, 16)
        zero = iota * 0

        def body(lb, carry):
            lanes = iota + 16 * lb
            for d in range(32):
                vals = plsc.load_gather(gbuf, [lanes, zero + d])
                obuf[d // 8, d % 8, pl.ds(16 * lb, 16)] = vals
            return carry

        lax.fori_loop(0, 8, body, 0)

    def write_out(j, q):
        beta = wid * n_blocks + j
        s = beta // 128
        kk = beta % 128
        for tr in range(4):
            pltpu.async_copy(obufs[q].at[tr], o5_hbm.at[s, tr, kk], wsems[q])

    def drain_write(q):
        for tr in range(4):
            pltpu.make_async_copy(obufs[q].at[tr], o5_hbm.at[0, tr, 0],
                                  wsems[q]).wait()

    for q in range(_NBUF):
        issue(q, q)

    def body(i, carry):
        for q in range(_NBUF):
            j = _NBUF * i + q
            drain_gather(q)

            @pl.when(i > 0)
            def _():
                drain_write(q)

            transpose_block(gbufs[q], obufs[q])
            write_out(j, q)

            @pl.when(j + _NBUF < n_blocks)
            def _():
                issue(j + _NBUF, q)

        return carry

    lax.fori_loop(0, n_blocks // _NBUF, body, 0)
    for q in range(_NBUF):
        drain_write(q)


def kernel(x, w):
    b, s_len = x.shape
    d = w.shape[1]
    n_blk_rows = b // _LANES
    total_blocks = s_len * n_blk_rows
    assert total_blocks % (_NBUF * _NW) == 0
    n_blocks = total_blocks // _NW
    idx = x.astype(jnp.int32).T.reshape(_NW, n_blocks, _LANES)

    mesh = plsc.VectorSubcoreMesh(core_axis_name="c", subcore_axis_name="s")
    k = functools.partial(
        pl.kernel,
        mesh=mesh,
        out_type=jax.ShapeDtypeStruct((s_len, d // 8, n_blk_rows, 8, _LANES),
                                      jnp.float32),
        scratch_types=(
            [pltpu.VMEM((n_blocks, _LANES), jnp.int32)]
            + [pltpu.VMEM((_LANES, d), jnp.float32)] * _NBUF
            + [pltpu.VMEM((d // 8, 8, _LANES), jnp.float32)] * _NBUF
            + [pltpu.SemaphoreType.DMA] * (2 * _NBUF)
        ),
        compiler_params=pltpu.CompilerParams(use_tc_tiling_on_sc=False,
                                             needs_layout_passes=False),
    )(functools.partial(_embed_kernel, n_blocks))
    o5 = k(idx, w)
    return o5.transpose(2, 4, 0, 1, 3).reshape(b, s_len, d)


# speedup vs baseline: 1.6718x; 1.5643x over previous
import functools

import jax
import jax.numpy as jnp
from jax import lax
from jax.experimental import pallas as pl
from jax.experimental.pallas import tpu as pltpu
from jax.experimental.pallas import tpu_sc as plsc

_info = plsc.get_sparse_core_info()
_NC, _NS = _info.num_cores, _info.num_subcores
_NW = _NC * _NS

_LANES = 128
_NBUF = 4


def _embed_kernel(n_blocks, idx_hbm, table_hbm, o5_hbm, idx_v,
                  g0, g1, g2, g3, ob0, ob1, ob2, ob3,
                  gs0, gs1, gs2, gs3, ws0, ws1, ws2, ws3):
    wid = lax.axis_index("s") * _NC + lax.axis_index("c")
    pltpu.sync_copy(idx_hbm.at[wid], idx_v)
    gbufs = (g0, g1, g2, g3)
    obufs = (ob0, ob1, ob2, ob3)
    gsems = (gs0, gs1, gs2, gs3)
    wsems = (ws0, ws1, ws2, ws3)

    def issue(j, q):
        pltpu.async_copy(table_hbm.at[idx_v.at[j]], gbufs[q], gsems[q])

    def drain_gather(q):
        pltpu.make_async_copy(table_hbm.at[idx_v.at[0]], gbufs[q],
                              gsems[q]).wait()

    def transpose_block(gbuf, obuf):
        iota = lax.iota(jnp.int32, 16)
        lvecs = [iota + 16 * lb for lb in range(8)]

        def body(c, carry):
            for lb in range(8):
                lvec = lvecs[lb]
                dvec = (lvec + c) & 31
                vals = plsc.load_gather(gbuf, [lvec, dvec])
                plsc.store_scatter(obuf, [dvec >> 3, dvec & 7, lvec], vals)
            return carry

        lax.fori_loop(0, 32, body, 0)

    def write_out(j, q):
        beta = wid * n_blocks + j
        s = beta // 128
        kk = beta % 128
        for tr in range(4):
            pltpu.async_copy(obufs[q].at[tr], o5_hbm.at[s, tr, kk], wsems[q])

    def drain_write(q):
        for tr in range(4):
            pltpu.make_async_copy(obufs[q].at[tr], o5_hbm.at[0, tr, 0],
                                  wsems[q]).wait()

    for q in range(_NBUF):
        issue(q, q)

    def body(i, carry):
        for q in range(_NBUF):
            j = _NBUF * i + q
            drain_gather(q)

            @pl.when(i > 0)
            def _():
                drain_write(q)

            transpose_block(gbufs[q], obufs[q])
            write_out(j, q)

            @pl.when(j + _NBUF < n_blocks)
            def _():
                issue(j + _NBUF, q)

        return carry

    lax.fori_loop(0, n_blocks // _NBUF, body, 0)
    for q in range(_NBUF):
        drain_write(q)


def kernel(x, w):
    b, s_len = x.shape
    d = w.shape[1]
    n_blk_rows = b // _LANES
    total_blocks = s_len * n_blk_rows
    assert total_blocks % (_NBUF * _NW) == 0
    n_blocks = total_blocks // _NW
    idx = x.astype(jnp.int32).T.reshape(_NW, n_blocks, _LANES)

    mesh = plsc.VectorSubcoreMesh(core_axis_name="c", subcore_axis_name="s")
    k = functools.partial(
        pl.kernel,
        mesh=mesh,
        out_type=jax.ShapeDtypeStruct((s_len, d // 8, n_blk_rows, 8, _LANES),
                                      jnp.float32),
        scratch_types=(
            [pltpu.VMEM((n_blocks, _LANES), jnp.int32)]
            + [pltpu.VMEM((_LANES, d), jnp.float32)] * _NBUF
            + [pltpu.VMEM((d // 8, 8, _LANES), jnp.float32)] * _NBUF
            + [pltpu.SemaphoreType.DMA] * (2 * _NBUF)
        ),
        compiler_params=pltpu.CompilerParams(use_tc_tiling_on_sc=False,
                                             needs_layout_passes=False),
    )(functools.partial(_embed_kernel, n_blocks))
    o5 = k(idx, w)
    return o5.transpose(2, 4, 0, 1, 3).reshape(b, s_len, d)


# speedup vs baseline: 2.3400x; 1.3997x over previous
import functools

import jax
import jax.numpy as jnp
from jax import lax
from jax.experimental import pallas as pl
from jax.experimental.pallas import tpu as pltpu
from jax.experimental.pallas import tpu_sc as plsc

_info = plsc.get_sparse_core_info()
_NC, _NS = _info.num_cores, _info.num_subcores
_NW = _NC * _NS

_LANES = 128
_NBUF = 4


def _transpose_128x32(src, dst, iota):
    lvecs = [iota + 16 * lb for lb in range(8)]

    def body(c, carry):
        for lb in range(8):
            lvec = lvecs[lb]
            dvec = (lvec + c) & 31
            vals = plsc.load_gather(src, [dvec, lvec])
            flat = lvec * 32 + dvec
            plsc.store_scatter(dst, [flat >> 7, flat & 127], vals)
        return carry

    lax.fori_loop(0, 32, body, 0)


def _tableprep_kernel(n_full, wt_hbm, wrm_hbm, b0, b1, t0, t1,
                      gs0, gs1, ws0, ws1):
    wid = lax.axis_index("s") * _NC + lax.axis_index("c")
    iota = lax.iota(jnp.int32, 16)
    bins = (b0, b1)
    touts = (t0, t1)
    gsems = (gs0, gs1)
    wsems = (ws0, ws1)

    def issue(k, q):
        off = pl.multiple_of(128 * k, 128)
        pltpu.async_copy(wt_hbm.at[:, pl.ds(off, 128)], bins[q], gsems[q])

    def drain_in(q):
        pltpu.make_async_copy(wt_hbm.at[:, pl.ds(0, 128)], bins[q],
                              gsems[q]).wait()

    def write(k, q):
        off = pl.multiple_of(32 * k, 32)
        pltpu.async_copy(touts[q], wrm_hbm.at[pl.ds(off, 32)], wsems[q])

    def drain_write(q):
        pltpu.make_async_copy(touts[q], wrm_hbm.at[pl.ds(0, 32)],
                              wsems[q]).wait()

    n_mine = (n_full - wid + _NW - 1) // _NW
    issue(wid, 0)
    issue(wid + _NW, 1)

    def body(i, carry):
        for q in range(2):
            j = 2 * i + q

            @pl.when(j < n_mine)
            def _():
                k = wid + _NW * j
                drain_in(q)

                @pl.when(j >= 2)
                def _():
                    drain_write(q)

                _transpose_128x32(bins[q], touts[q], iota)
                write(k, q)

                @pl.when(j + 2 < n_mine)
                def _():
                    issue(k + 2 * _NW, q)

        return carry

    lax.fori_loop(0, (n_mine + 1) // 2, body, 0)
    drain_write(0)
    drain_write(1)


def _embed_kernel(n_blocks, tail_base, idx_hbm, table_hbm, wtail_hbm, o5_hbm,
                  idx_v, wtail_v, g0, g1, g2, g3, ob0, ob1, ob2, ob3,
                  gs0, gs1, gs2, gs3, ws0, ws1, ws2, ws3):
    wid = lax.axis_index("s") * _NC + lax.axis_index("c")
    pltpu.sync_copy(idx_hbm.at[wid], idx_v)
    pltpu.sync_copy(wtail_hbm, wtail_v)
    gbufs = (g0, g1, g2, g3)
    obufs = (ob0, ob1, ob2, ob3)
    gsems = (gs0, gs1, gs2, gs3)
    wsems = (ws0, ws1, ws2, ws3)

    def issue(j, q):
        pltpu.async_copy(table_hbm.at[idx_v.at[j]], gbufs[q], gsems[q])

    def drain_gather(q):
        pltpu.make_async_copy(table_hbm.at[idx_v.at[0]], gbufs[q],
                              gsems[q]).wait()

    def transpose_block(gbuf, obuf):
        iota = lax.iota(jnp.int32, 16)
        lvecs = [iota + 16 * lb for lb in range(8)]

        def body(c, carry):
            for lb in range(8):
                lvec = lvecs[lb]
                dvec = (lvec + c) & 31
                vals = plsc.load_gather(gbuf, [lvec, dvec])
                plsc.store_scatter(obuf, [dvec >> 3, dvec & 7, lvec], vals)
            return carry

        lax.fori_loop(0, 32, body, 0)

    def write_out(j, q):
        beta = wid * n_blocks + j
        s = beta // 128
        kk = beta % 128
        for tr in range(4):
            pltpu.async_copy(obufs[q].at[tr], o5_hbm.at[s, tr, kk], wsems[q])

    def drain_write(q):
        for tr in range(4):
            pltpu.make_async_copy(obufs[q].at[tr], o5_hbm.at[0, tr, 0],
                                  wsems[q]).wait()

    for q in range(_NBUF):
        issue(q, q)

    def patch_tail(j, obuf):
        iota = lax.iota(jnp.int32, 16)

        def pbody(g, carry):
            iv = idx_v[j, pl.ds(16 * g, 16)]
            msk = iv >= tail_base
            tix = jnp.minimum(jnp.maximum(iv - tail_base, 0), 63)

            def dbody(d, carry2):
                tv = plsc.load_gather(wtail_v, [tix, (iota & 0) + d])
                ov = obuf[d // 8, d % 8, pl.ds(16 * g, 16)]
                obuf[d // 8, d % 8, pl.ds(16 * g, 16)] = jnp.where(
                    msk, tv, ov)
                return carry2

            lax.fori_loop(0, 32, dbody, 0)
            return carry

        lax.fori_loop(0, 8, pbody, 0)

    def block_max(j):
        m = idx_v[j, pl.ds(0, 16)]
        for g in range(1, 8):
            m = jnp.maximum(m, idx_v[j, pl.ds(16 * g, 16)])
        return lax.reduce_max(m, (0,))

    def body(i, carry):
        for q in range(_NBUF):
            j = _NBUF * i + q
            drain_gather(q)

            @pl.when(i > 0)
            def _():
                drain_write(q)

            transpose_block(gbufs[q], obufs[q])

            @pl.when(block_max(j) >= tail_base)
            def _():
                patch_tail(j, obufs[q])

            write_out(j, q)

            @pl.when(j + _NBUF < n_blocks)
            def _():
                issue(j + _NBUF, q)

        return carry

    lax.fori_loop(0, n_blocks // _NBUF, body, 0)
    for q in range(_NBUF):
        drain_write(q)


def kernel(x, w):
    b, s_len = x.shape
    n_emb, d = w.shape
    n_blk_rows = b // _LANES
    total_blocks = s_len * n_blk_rows
    assert total_blocks % (_NBUF * _NW) == 0
    n_blocks = total_blocks // _NW
    idx = x.astype(jnp.int32).T.reshape(_NW, n_blocks, _LANES)

    mesh = plsc.VectorSubcoreMesh(core_axis_name="c", subcore_axis_name="s")

    n_full = n_emb // _LANES
    tail_base = n_full * _LANES
    n_pad_rows = (n_full + 1) * _LANES * d // _LANES
    prep = functools.partial(
        pl.kernel,
        mesh=mesh,
        out_type=jax.ShapeDtypeStruct((n_pad_rows, _LANES), jnp.float32),
        scratch_types=(
            [pltpu.VMEM((d, _LANES), jnp.float32)] * 2
            + [pltpu.VMEM((d, _LANES), jnp.float32)] * 2
            + [pltpu.SemaphoreType.DMA] * 4
        ),
        compiler_params=pltpu.CompilerParams(use_tc_tiling_on_sc=True,
                                             needs_layout_passes=False),
    )(functools.partial(_tableprep_kernel, n_full))
    wrm = prep(w.T).reshape(n_pad_rows * _LANES // d, d)
    wtail = w[tail_base:]

    k = functools.partial(
        pl.kernel,
        mesh=mesh,
        out_type=jax.ShapeDtypeStruct((s_len, d // 8, n_blk_rows, 8, _LANES),
                                      jnp.float32),
        scratch_types=(
            [pltpu.VMEM((n_blocks, _LANES), jnp.int32)]
            + [pltpu.VMEM((n_emb - tail_base, d), jnp.float32)]
            + [pltpu.VMEM((_LANES, d), jnp.float32)] * _NBUF
            + [pltpu.VMEM((d // 8, 8, _LANES), jnp.float32)] * _NBUF
            + [pltpu.SemaphoreType.DMA] * (2 * _NBUF)
        ),
        compiler_params=pltpu.CompilerParams(use_tc_tiling_on_sc=False,
                                             needs_layout_passes=False),
    )(functools.partial(_embed_kernel, n_blocks, tail_base))
    o5 = k(idx, wrm, wtail)
    return o5.transpose(2, 4, 0, 1, 3).reshape(b, s_len, d)


# speedup vs baseline: 2.3695x; 1.0126x over previous
import functools

import jax
import jax.numpy as jnp
from jax import lax
from jax.experimental import pallas as pl
from jax.experimental.pallas import tpu as pltpu
from jax.experimental.pallas import tpu_sc as plsc

_info = plsc.get_sparse_core_info()
_NC, _NS = _info.num_cores, _info.num_subcores
_NW = _NC * _NS

_LANES = 128
_NBUF = 4


def _transpose_128x32(src, dst, iota):
    lvecs = [iota + 16 * lb for lb in range(8)]
    l32s = [lv * 32 for lv in lvecs]

    def body(c, carry):
        for cc in (2 * c, 2 * c + 1):
            for lb in range(8):
                lvec = lvecs[lb]
                dvec = (lvec + cc) & 31
                vals = plsc.load_gather(src, [dvec, lvec])
                plsc.store_scatter(dst, [l32s[lb] + dvec], vals)
        return carry

    lax.fori_loop(0, 16, body, 0)


def _tableprep_kernel(n_full, wt_hbm, wrm_hbm, b0, b1, t0, t1,
                      gs0, gs1, ws0, ws1):
    wid = lax.axis_index("s") * _NC + lax.axis_index("c")
    iota = lax.iota(jnp.int32, 16)
    bins = (b0, b1)
    touts = (t0, t1)
    gsems = (gs0, gs1)
    wsems = (ws0, ws1)

    def issue(k, q):
        off = pl.multiple_of(128 * k, 128)
        pltpu.async_copy(wt_hbm.at[:, pl.ds(off, 128)], bins[q], gsems[q])

    def drain_in(q):
        pltpu.make_async_copy(wt_hbm.at[:, pl.ds(0, 128)], bins[q],
                              gsems[q]).wait()

    def write(k, q):
        off = pl.multiple_of(4096 * k, 4096)
        pltpu.async_copy(touts[q], wrm_hbm.at[pl.ds(off, 4096)], wsems[q])

    def drain_write(q):
        pltpu.make_async_copy(touts[q], wrm_hbm.at[pl.ds(0, 4096)],
                              wsems[q]).wait()

    n_mine = (n_full - wid + _NW - 1) // _NW
    issue(wid, 0)
    issue(wid + _NW, 1)

    def body(i, carry):
        for q in range(2):
            j = 2 * i + q

            @pl.when(j < n_mine)
            def _():
                k = wid + _NW * j
                drain_in(q)

                @pl.when(j >= 2)
                def _():
                    drain_write(q)

                _transpose_128x32(bins[q], touts[q], iota)
                write(k, q)

                @pl.when(j + 2 < n_mine)
                def _():
                    issue(k + 2 * _NW, q)

        return carry

    lax.fori_loop(0, (n_mine + 1) // 2, body, 0)
    drain_write(0)
    drain_write(1)


def _embed_kernel(n_blocks, tail_base, idx_hbm, table_hbm, wtail_hbm, o5_hbm,
                  idx_v, wtail_v, g0, g1, g2, g3, ob0, ob1, ob2, ob3,
                  gs0, gs1, gs2, gs3, ws0, ws1, ws2, ws3):
    wid = lax.axis_index("s") * _NC + lax.axis_index("c")
    pltpu.sync_copy(idx_hbm.at[wid], idx_v)
    pltpu.sync_copy(wtail_hbm, wtail_v)
    gbufs = (g0, g1, g2, g3)
    obufs = (ob0, ob1, ob2, ob3)
    gsems = (gs0, gs1, gs2, gs3)
    wsems = (ws0, ws1, ws2, ws3)

    def issue(j, q):
        pltpu.async_copy(table_hbm.at[idx_v.at[j]], gbufs[q], gsems[q])

    def drain_gather(q):
        pltpu.make_async_copy(table_hbm.at[idx_v.at[0]], gbufs[q],
                              gsems[q]).wait()

    def transpose_block(gbuf, obuf):
        iota = lax.iota(jnp.int32, 16)
        lvecs = [iota + 16 * lb for lb in range(8)]

        def body(c, carry):
            for cc in (2 * c, 2 * c + 1):
                for lb in range(8):
                    lvec = lvecs[lb]
                    dvec = (lvec + cc) & 31
                    vals = plsc.load_gather(gbuf, [lvec, dvec])
                    plsc.store_scatter(obuf, [dvec * 128 + lvec], vals)
            return carry

        lax.fori_loop(0, 16, body, 0)

    def write_out(j, q):
        beta = wid * n_blocks + j
        s = beta // 128
        kk = beta % 128
        for tr in range(4):
            pltpu.async_copy(obufs[q].at[pl.ds(1024 * tr, 1024)],
                             o5_hbm.at[s, tr, kk], wsems[q])

    def drain_write(q):
        for tr in range(4):
            pltpu.make_async_copy(obufs[q].at[pl.ds(1024 * tr, 1024)],
                                  o5_hbm.at[0, tr, 0], wsems[q]).wait()

    for q in range(_NBUF):
        issue(q, q)

    def patch_tail(j, obuf):
        iota = lax.iota(jnp.int32, 16)

        def pbody(g, carry):
            iv = idx_v[j, pl.ds(16 * g, 16)]
            msk = iv >= tail_base
            tix = jnp.minimum(jnp.maximum(iv - tail_base, 0), 63)

            def dbody(d, carry2):
                tv = plsc.load_gather(wtail_v, [tix, (iota & 0) + d])
                off = d * 128 + 16 * g
                ov = obuf[pl.ds(off, 16)]
                obuf[pl.ds(off, 16)] = jnp.where(msk, tv, ov)
                return carry2

            lax.fori_loop(0, 32, dbody, 0)
            return carry

        lax.fori_loop(0, 8, pbody, 0)

    def block_max(j):
        m = idx_v[j, pl.ds(0, 16)]
        for g in range(1, 8):
            m = jnp.maximum(m, idx_v[j, pl.ds(16 * g, 16)])
        return lax.reduce_max(m, (0,))

    def body(i, carry):
        for q in range(_NBUF):
            j = _NBUF * i + q
            drain_gather(q)

            @pl.when(i > 0)
            def _():
                drain_write(q)

            transpose_block(gbufs[q], obufs[q])

            @pl.when(block_max(j) >= tail_base)
            def _():
                patch_tail(j, obufs[q])

            write_out(j, q)

            @pl.when(j + _NBUF < n_blocks)
            def _():
                issue(j + _NBUF, q)

        return carry

    lax.fori_loop(0, n_blocks // _NBUF, body, 0)
    for q in range(_NBUF):
        drain_write(q)


def kernel(x, w):
    b, s_len = x.shape
    n_emb, d = w.shape
    n_blk_rows = b // _LANES
    total_blocks = s_len * n_blk_rows
    assert total_blocks % (_NBUF * _NW) == 0
    n_blocks = total_blocks // _NW
    idx = x.astype(jnp.int32).T.reshape(_NW, n_blocks, _LANES)

    mesh = plsc.VectorSubcoreMesh(core_axis_name="c", subcore_axis_name="s")

    n_full = n_emb // _LANES
    tail_base = n_full * _LANES
    n_pad_rows = (n_full + 1) * _LANES * d // _LANES
    prep = functools.partial(
        pl.kernel,
        mesh=mesh,
        out_type=jax.ShapeDtypeStruct((n_pad_rows * _LANES,), jnp.float32),
        scratch_types=(
            [pltpu.VMEM((d, _LANES), jnp.float32)] * 2
            + [pltpu.VMEM((d * _LANES,), jnp.float32)] * 2
            + [pltpu.SemaphoreType.DMA] * 4
        ),
        compiler_params=pltpu.CompilerParams(use_tc_tiling_on_sc=True,
                                             needs_layout_passes=False),
    )(functools.partial(_tableprep_kernel, n_full))
    wrm = prep(w.T).reshape(n_pad_rows * _LANES // d, d)
    wtail = w[tail_base:]

    k = functools.partial(
        pl.kernel,
        mesh=mesh,
        out_type=jax.ShapeDtypeStruct((s_len, d // 8, n_blk_rows, 8 * _LANES),
                                      jnp.float32),
        scratch_types=(
            [pltpu.VMEM((n_blocks, _LANES), jnp.int32)]
            + [pltpu.VMEM((n_emb - tail_base, d), jnp.float32)]
            + [pltpu.VMEM((_LANES, d), jnp.float32)] * _NBUF
            + [pltpu.VMEM((d * _LANES,), jnp.float32)] * _NBUF
            + [pltpu.SemaphoreType.DMA] * (2 * _NBUF)
        ),
        compiler_params=pltpu.CompilerParams(use_tc_tiling_on_sc=False,
                                             needs_layout_passes=False),
    )(functools.partial(_embed_kernel, n_blocks, tail_base))
    o5 = k(idx, wrm, wtail)
    o5 = o5.reshape(s_len, d // 8, n_blk_rows, 8, _LANES)
    return o5.transpose(2, 4, 0, 1, 3).reshape(b, s_len, d)
